# Initial kernel scaffold; baseline (speedup 1.0000x reference)
import functools

import jax
import jax.numpy as jnp
from jax import lax
from jax.experimental import pallas as pl
from jax.experimental.pallas import tpu as pltpu
from jax.experimental.pallas import tpu_sc as plsc

N = 50000
E = 800000
HID = 64
H = 4
C = 64
ED = 4

NC = 2
NS = 16
PARTC = C // NC
PAD = 16

CHUNK = 128
NCHUNKS = E // CHUNK
ROWS_PER_TILE = N // NS
ZROWS = 125

_mesh = plsc.VectorSubcoreMesh(
    core_axis_name="c", subcore_axis_name="s", num_cores=NC, num_subcores=NS)


def _proj_body(h_ref, wp_ref, ws_ref, wd_ref, x2_ref, asrc_ref, adst_ref):
    hb = h_ref[...]
    xb = jnp.dot(hb, wp_ref[...], preferred_element_type=jnp.float32)
    x2_ref[0] = xb[:, :128]
    x2_ref[1] = xb[:, 128:]
    asrc_ref[...] = jnp.dot(hb, ws_ref[...], preferred_element_type=jnp.float32)
    adst_ref[...] = jnp.dot(hb, wd_ref[...], preferred_element_type=jnp.float32)


_proj = pl.pallas_call(
    _proj_body,
    grid=(125,),
    in_specs=[
        pl.BlockSpec((400, HID), lambda i: (i, 0)),
        pl.BlockSpec((HID, H * C), lambda i: (0, 0)),
        pl.BlockSpec((HID, PAD), lambda i: (0, 0)),
        pl.BlockSpec((HID, PAD), lambda i: (0, 0)),
    ],
    out_specs=[
        pl.BlockSpec((NC, 400, 128), lambda i: (0, i, 0)),
        pl.BlockSpec((400, PAD), lambda i: (i, 0)),
        pl.BlockSpec((400, PAD), lambda i: (i, 0)),
    ],
    out_shape=[
        jax.ShapeDtypeStruct((NC, N, 128), jnp.float32),
        jax.ShapeDtypeStruct((N, PAD), jnp.float32),
        jax.ShapeDtypeStruct((N, PAD), jnp.float32),
    ],
)


def _ae_body(ea_ref, w_ref, ae_ref):
    ea = ea_ref[...]
    w = w_ref[...]
    acc = ea[:, 0:1] * w[0:1, :]
    for d in range(1, ED):
        acc = acc + ea[:, d:d + 1] * w[d:d + 1, :]
    ae_ref[...] = acc


_ae_proj = pl.pallas_call(
    _ae_body,
    grid=(500,),
    in_specs=[
        pl.BlockSpec((1600, ED), lambda i: (i, 0)),
        pl.BlockSpec((ED, PAD), lambda i: (0, 0)),
    ],
    out_specs=pl.BlockSpec((1600, PAD), lambda i: (i, 0)),
    out_shape=jax.ShapeDtypeStruct((E, PAD), jnp.float32),
)


@functools.partial(
    pl.kernel,
    mesh=_mesh,
    out_type=(
        jax.ShapeDtypeStruct((E, PAD), jnp.float32),
        jax.ShapeDtypeStruct((NC * N, PAD), jnp.float32),
    ),
    scratch_types=[
        pltpu.VMEM((CHUNK, PAD), jnp.float32),
        pltpu.VMEM((CHUNK, PAD), jnp.float32),
        pltpu.VMEM((CHUNK, PAD), jnp.float32),
        pltpu.VMEM((CHUNK, PAD), jnp.float32),
        pltpu.VMEM((CHUNK,), jnp.int32),
        pltpu.VMEM((CHUNK,), jnp.int32),
        pltpu.VMEM((ZROWS, PAD), jnp.float32),
        pltpu.VMEM_SHARED((N, PAD), jnp.float32),
    ],
)
def _edge_ex(src_hbm, dst_hbm, asrc_hbm, adst_hbm, ae_hbm,
             ex_hbm, dpart_hbm,
             As, Ad, aeb, exb, sbuf, dbuf, zb, den_sp):
    c = lax.axis_index("c")
    s = lax.axis_index("s")
    wid = s * NC + c

    def _zrow(r, carry):
        zb[r] = jnp.zeros((PAD,), jnp.float32)
        return carry
    lax.fori_loop(0, ZROWS, _zrow, 0)

    def _zcp(i, carry):
        pltpu.sync_copy(zb, den_sp.at[pl.ds(s * ROWS_PER_TILE + i * ZROWS, ZROWS)])
        return carry
    lax.fori_loop(0, ROWS_PER_TILE // ZROWS, _zcp, 0)
    plsc.subcore_barrier()

    nch = jnp.where(wid < 10, 196, 195)
    base_ch = wid * 195 + jnp.minimum(wid, 10)

    def _chunk(i, carry):
        base = (base_ch + i) * CHUNK
        pltpu.sync_copy(src_hbm.at[pl.ds(base, CHUNK)], sbuf)
        pltpu.sync_copy(dst_hbm.at[pl.ds(base, CHUNK)], dbuf)
        pltpu.sync_copy(ae_hbm.at[pl.ds(base, CHUNK)], aeb)
        pltpu.sync_copy(asrc_hbm.at[sbuf], As)
        pltpu.sync_copy(adst_hbm.at[dbuf], Ad)

        def _edge(e, ecarry):
            sv = As[e] + Ad[e] + aeb[e]
            sv = jnp.maximum(sv, 0.2 * sv)
            exb[e] = jnp.exp(sv)
            return ecarry
        lax.fori_loop(0, CHUNK, _edge, 0)

        pltpu.sync_copy(exb, ex_hbm.at[pl.ds(base, CHUNK)])
        pltpu.sync_copy(exb, den_sp.at[dbuf], add=True)
        return carry
    lax.fori_loop(0, nch, _chunk, 0)

    plsc.subcore_barrier()
    pltpu.sync_copy(
        den_sp.at[pl.ds(s * ROWS_PER_TILE, ROWS_PER_TILE)],
        dpart_hbm.at[pl.ds(c * N + s * ROWS_PER_TILE, ROWS_PER_TILE)])


def _rden_body(dp_ref, rden_ref):
    rden_ref[...] = 1.0 / (dp_ref[0] + dp_ref[1] + 1e-16)


_rden = pl.pallas_call(
    _rden_body,
    grid=(1,),
    in_specs=[pl.BlockSpec((NC, N * PAD // 128, 128), lambda i: (0, 0, 0))],
    out_specs=pl.BlockSpec((N * PAD // 128, 128), lambda i: (0, 0)),
    out_shape=jax.ShapeDtypeStruct((N * PAD // 128, 128), jnp.float32),
)


@functools.partial(
    pl.kernel,
    mesh=_mesh,
    out_type=jax.ShapeDtypeStruct((NC * N, PARTC), jnp.float32),
    scratch_types=[
        pltpu.VMEM((CHUNK, 128), jnp.float32),
        pltpu.VMEM((CHUNK, PARTC), jnp.float32),
        pltpu.VMEM((CHUNK, PAD), jnp.float32),
        pltpu.VMEM((CHUNK, PAD), jnp.float32),
        pltpu.VMEM((CHUNK, PAD), jnp.float32),
        pltpu.VMEM((CHUNK,), jnp.int32),
        pltpu.VMEM((CHUNK,), jnp.int32),
        pltpu.VMEM((CHUNK,), jnp.int32),
        pltpu.VMEM((ZROWS, PARTC), jnp.float32),
        pltpu.VMEM_SHARED((N, PARTC), jnp.float32),
    ],
)
def _aggregate(src_hbm, dst_hbm, x2_hbm, rden_hbm, ex_hbm,
               agg_hbm,
               X, M, R, exb, attb, sbuf, dbuf, ibuf, zb, agg_sp):
    c = lax.axis_index("c")
    s = lax.axis_index("s")

    def _zrow(r, carry):
        zb[r, pl.ds(0, 16)] = jnp.zeros((16,), jnp.float32)
        zb[r, pl.ds(16, 16)] = jnp.zeros((16,), jnp.float32)
        return carry
    lax.fori_loop(0, ZROWS, _zrow, 0)

    def _zcp(i, carry):
        pltpu.sync_copy(zb, agg_sp.at[pl.ds(s * ROWS_PER_TILE + i * ZROWS, ZROWS)])
        return carry
    lax.fori_loop(0, ROWS_PER_TILE // ZROWS, _zcp, 0)
    plsc.subcore_barrier()

    nch = jnp.where(s < 10, 391, 390)
    base_ch = s * 390 + jnp.minimum(s, 10)
    cN = c * N

    def _chunk(i, carry):
        base = (base_ch + i) * CHUNK
        pltpu.sync_copy(src_hbm.at[pl.ds(base, CHUNK)], sbuf)
        pltpu.sync_copy(dst_hbm.at[pl.ds(base, CHUNK)], dbuf)

        def _gidx(g, gcarry):
            ibuf[pl.ds(g * 16, 16)] = sbuf[pl.ds(g * 16, 16)] + cN
            return gcarry
        lax.fori_loop(0, CHUNK // 16, _gidx, 0)

        pltpu.sync_copy(x2_hbm.at[ibuf], X)
        pltpu.sync_copy(rden_hbm.at[dbuf], R)
        pltpu.sync_copy(ex_hbm.at[pl.ds(base, CHUNK)], exb)

        def _edge(e, ecarry):
            attb[e] = exb[e] * R[e]
            a0 = attb[e, 0]
            a1 = attb[e, 1]
            a2 = attb[e, 2]
            a3 = attb[e, 3]
            for q in range(2):
                m = (a0 * X[e, pl.ds(q * 16, 16)]
                     + a1 * X[e, pl.ds(32 + q * 16, 16)]
                     + a2 * X[e, pl.ds(64 + q * 16, 16)]
                     + a3 * X[e, pl.ds(96 + q * 16, 16)])
                M[e, pl.ds(q * 16, 16)] = m
            return ecarry
        lax.fori_loop(0, CHUNK, _edge, 0)

        pltpu.sync_copy(M, agg_sp.at[dbuf], add=True)
        return carry
    lax.fori_loop(0, nch, _chunk, 0)

    plsc.subcore_barrier()
    pltpu.sync_copy(
        agg_sp.at[pl.ds(s * ROWS_PER_TILE, ROWS_PER_TILE)],
        agg_hbm.at[pl.ds(cN + s * ROWS_PER_TILE, ROWS_PER_TILE)])


def _fin_body(agg_ref, bias_ref, gam_ref, bet_ref, y_ref):
    a = jnp.concatenate([agg_ref[0], agg_ref[1]], axis=-1) * (1.0 / H)
    a = a + bias_ref[...]
    mu = jnp.mean(a, axis=-1, keepdims=True)
    var = jnp.mean((a - mu) ** 2, axis=-1, keepdims=True)
    yn = (a - mu) / jnp.sqrt(var + 1e-5) * gam_ref[...] + bet_ref[...]
    y_ref[...] = yn * jax.nn.sigmoid(yn)


_finalize = pl.pallas_call(
    _fin_body,
    grid=(125,),
    in_specs=[
        pl.BlockSpec((NC, 400, PARTC), lambda i: (0, i, 0)),
        pl.BlockSpec((1, C), lambda i: (0, 0)),
        pl.BlockSpec((1, C), lambda i: (0, 0)),
        pl.BlockSpec((1, C), lambda i: (0, 0)),
    ],
    out_specs=pl.BlockSpec((400, C), lambda i: (i, 0)),
    out_shape=jax.ShapeDtypeStruct((N, C), jnp.float32),
)


def kernel(t, h, edge_index, edge_attr, W, We, att_src, att_dst, att_edge,
           bias, ln_gamma, ln_beta):
    del t
    f32 = jnp.float32
    Wr = W.reshape(HID, H, NC, PARTC)
    Wp = Wr.transpose(0, 2, 1, 3).reshape(HID, H * C)
    Ws = jnp.einsum("khc,hc->kh", W.reshape(HID, H, C), att_src)
    Wd = jnp.einsum("khc,hc->kh", W.reshape(HID, H, C), att_dst)
    Ae = jnp.einsum("dhc,hc->dh", We.reshape(ED, H, C), att_edge)
    Ws16 = jnp.zeros((HID, PAD), f32).at[:, :H].set(Ws)
    Wd16 = jnp.zeros((HID, PAD), f32).at[:, :H].set(Wd)
    Ae16 = jnp.zeros((ED, PAD), f32).at[:, :H].set(Ae)

    x2, asrc16, adst16 = _proj(h, Wp, Ws16, Wd16)
    ae16 = _ae_proj(edge_attr, Ae16)

    src = edge_index[0]
    dst = edge_index[1]

    ex16, dpart = _edge_ex(src, dst, asrc16, adst16, ae16)
    rden = _rden(dpart.reshape(NC, N * PAD // 128, 128)).reshape(N, PAD)
    agg = _aggregate(src, dst, x2.reshape(NC * N, 128), rden, ex16)
    y = _finalize(agg.reshape(NC, N, PARTC), bias.reshape(1, C),
                  ln_gamma.reshape(1, C), ln_beta.reshape(1, C))
    return y


# speedup vs baseline: 21.2064x; 21.2064x over previous
import functools

import jax
import jax.numpy as jnp
from jax import lax
from jax.experimental import pallas as pl
from jax.experimental.pallas import tpu as pltpu
from jax.experimental.pallas import tpu_sc as plsc

N = 50000
E = 800000
HID = 64
H = 4
C = 64
ED = 4

NC = 2
NS = 16
PARTC = C // NC
PAD = 16

CHUNK = 128
NCHUNKS = E // CHUNK
ZCH = 200
NZCH = N // ZCH

_mesh = plsc.VectorSubcoreMesh(
    core_axis_name="c", subcore_axis_name="s", num_cores=NC, num_subcores=NS)

_sc_params = pltpu.CompilerParams(use_tc_tiling_on_sc=False)


def _proj_body(h_ref, wp_ref, ws_ref, wd_ref, x2_ref, asrc_ref, adst_ref):
    hb = h_ref[...]
    xb = jnp.dot(hb, wp_ref[...], preferred_element_type=jnp.float32)
    x2_ref[0] = xb[:, :128]
    x2_ref[1] = xb[:, 128:]
    asrc_ref[...] = jnp.dot(hb, ws_ref[...], preferred_element_type=jnp.float32)
    adst_ref[...] = jnp.dot(hb, wd_ref[...], preferred_element_type=jnp.float32)


_proj = pl.pallas_call(
    _proj_body,
    grid=(125,),
    in_specs=[
        pl.BlockSpec((400, HID), lambda i: (i, 0)),
        pl.BlockSpec((HID, H * C), lambda i: (0, 0)),
        pl.BlockSpec((HID, PAD), lambda i: (0, 0)),
        pl.BlockSpec((HID, PAD), lambda i: (0, 0)),
    ],
    out_specs=[
        pl.BlockSpec((NC, 400, 128), lambda i: (0, i, 0)),
        pl.BlockSpec((400, PAD), lambda i: (i, 0)),
        pl.BlockSpec((400, PAD), lambda i: (i, 0)),
    ],
    out_shape=[
        jax.ShapeDtypeStruct((NC, N, 128), jnp.float32),
        jax.ShapeDtypeStruct((N, PAD), jnp.float32),
        jax.ShapeDtypeStruct((N, PAD), jnp.float32),
    ],
)


def _ae_body(ea_ref, w_ref, ae_ref):
    ea = ea_ref[...]
    w = w_ref[...]
    acc = ea[:, 0:1] * w[0:1, :]
    for d in range(1, ED):
        acc = acc + ea[:, d:d + 1] * w[d:d + 1, :]
    ae_ref[...] = acc


_ae_proj = pl.pallas_call(
    _ae_body,
    grid=(500,),
    in_specs=[
        pl.BlockSpec((1600, ED), lambda i: (i, 0)),
        pl.BlockSpec((ED, PAD), lambda i: (0, 0)),
    ],
    out_specs=pl.BlockSpec((1600, PAD), lambda i: (i, 0)),
    out_shape=jax.ShapeDtypeStruct((E, PAD), jnp.float32),
)


@functools.partial(
    pl.kernel,
    mesh=_mesh,
    out_type=(
        jax.ShapeDtypeStruct((E, PAD), jnp.float32),
        jax.ShapeDtypeStruct((NC * N, PAD), jnp.float32),
    ),
    scratch_types=[
        pltpu.VMEM((CHUNK, PAD), jnp.float32),
        pltpu.VMEM((CHUNK, PAD), jnp.float32),
        pltpu.VMEM((CHUNK, PAD), jnp.float32),
        pltpu.VMEM((CHUNK, PAD), jnp.float32),
        pltpu.VMEM((CHUNK,), jnp.int32),
        pltpu.VMEM((CHUNK,), jnp.int32),
        pltpu.VMEM((ZCH, PAD), jnp.float32),
        pltpu.VMEM_SHARED((N, PAD), jnp.float32),
    ],
    compiler_params=_sc_params,
)
def _edge_ex(src_hbm, dst_hbm, asrc_hbm, adst_hbm, ae_hbm,
             ex_hbm, dpart_hbm,
             As, Ad, aeb, exb, sbuf, dbuf, zb, den_sp):
    c = lax.axis_index("c")
    s = lax.axis_index("s")
    wid = s * NC + c

    def _zrow(r, carry):
        zb[r] = jnp.zeros((PAD,), jnp.float32)
        return carry
    lax.fori_loop(0, ZCH, _zrow, 0)

    nz = jnp.where(s < 10, 16, 15)
    base_z = s * 15 + jnp.minimum(s, 10)

    def _zcp(i, carry):
        pltpu.sync_copy(zb, den_sp.at[pl.ds((base_z + i) * ZCH, ZCH)])
        return carry
    lax.fori_loop(0, nz, _zcp, 0)
    plsc.subcore_barrier()

    nch = jnp.where(wid < 10, 196, 195)
    base_ch = wid * 195 + jnp.minimum(wid, 10)

    def _chunk(i, carry):
        base = (base_ch + i) * CHUNK
        pltpu.sync_copy(src_hbm.at[pl.ds(base, CHUNK)], sbuf)
        pltpu.sync_copy(dst_hbm.at[pl.ds(base, CHUNK)], dbuf)
        pltpu.sync_copy(ae_hbm.at[pl.ds(base, CHUNK)], aeb)
        pltpu.sync_copy(asrc_hbm.at[sbuf], As)
        pltpu.sync_copy(adst_hbm.at[dbuf], Ad)

        def _edge(e, ecarry):
            sv = As[e] + Ad[e] + aeb[e]
            sv = jnp.maximum(sv, 0.2 * sv)
            exb[e] = jnp.exp(sv)
            return ecarry
        lax.fori_loop(0, CHUNK, _edge, 0)

        pltpu.sync_copy(exb, ex_hbm.at[pl.ds(base, CHUNK)])
        pltpu.sync_copy(exb, den_sp.at[dbuf], add=True)
        return carry
    lax.fori_loop(0, nch, _chunk, 0)

    plsc.subcore_barrier()

    def _ocp(i, carry):
        r0 = (base_z + i) * ZCH
        pltpu.sync_copy(den_sp.at[pl.ds(r0, ZCH)],
                        dpart_hbm.at[pl.ds(c * N + r0, ZCH)])
        return carry
    lax.fori_loop(0, nz, _ocp, 0)


def _rden_body(dp_ref, rden_ref):
    rden_ref[...] = 1.0 / (dp_ref[0] + dp_ref[1] + 1e-16)


_rden = pl.pallas_call(
    _rden_body,
    grid=(1,),
    in_specs=[pl.BlockSpec((NC, N * PAD // 128, 128), lambda i: (0, 0, 0))],
    out_specs=pl.BlockSpec((N * PAD // 128, 128), lambda i: (0, 0)),
    out_shape=jax.ShapeDtypeStruct((N * PAD // 128, 128), jnp.float32),
)


@functools.partial(
    pl.kernel,
    mesh=_mesh,
    out_type=jax.ShapeDtypeStruct((NC * N, PARTC), jnp.float32),
    scratch_types=[
        pltpu.VMEM((CHUNK, 128), jnp.float32),
        pltpu.VMEM((CHUNK, PARTC), jnp.float32),
        pltpu.VMEM((CHUNK, PAD), jnp.float32),
        pltpu.VMEM((CHUNK, PAD), jnp.float32),
        pltpu.VMEM((CHUNK,), jnp.int32),
        pltpu.VMEM((CHUNK,), jnp.int32),
        pltpu.VMEM((CHUNK,), jnp.int32),
        pltpu.VMEM_SHARED((N, PARTC), jnp.float32),
    ],
    compiler_params=_sc_params,
)
def _aggregate(src_hbm, dst_hbm, x2_hbm, rden_hbm, ex_hbm,
               agg_hbm,
               X, M, R, exb, sbuf, dbuf, ibuf, agg_sp):
    c = lax.axis_index("c")
    s = lax.axis_index("s")

    def _zrow(r, carry):
        M[r, pl.ds(0, 16)] = jnp.zeros((16,), jnp.float32)
        M[r, pl.ds(16, 16)] = jnp.zeros((16,), jnp.float32)
        return carry
    lax.fori_loop(0, CHUNK, _zrow, 0)

    def _zcp(i, carry):
        pltpu.sync_copy(M.at[pl.ds(0, 125)],
                        agg_sp.at[pl.ds((s * 25 + i) * 125, 125)])
        return carry
    lax.fori_loop(0, 25, _zcp, 0)
    plsc.subcore_barrier()

    nz = jnp.where(s < 10, 16, 15)
    base_z = s * 15 + jnp.minimum(s, 10)

    nch = jnp.where(s < 10, 391, 390)
    base_ch = s * 390 + jnp.minimum(s, 10)
    cN = c * N

    def _chunk(i, carry):
        base = (base_ch + i) * CHUNK
        pltpu.sync_copy(src_hbm.at[pl.ds(base, CHUNK)], sbuf)
        pltpu.sync_copy(dst_hbm.at[pl.ds(base, CHUNK)], dbuf)

        def _gidx(g, gcarry):
            ibuf[pl.ds(g * 16, 16)] = sbuf[pl.ds(g * 16, 16)] + cN
            return gcarry
        lax.fori_loop(0, CHUNK // 16, _gidx, 0)

        pltpu.sync_copy(x2_hbm.at[ibuf], X)
        pltpu.sync_copy(rden_hbm.at[dbuf], R)
        pltpu.sync_copy(ex_hbm.at[pl.ds(base, CHUNK)], exb)

        def _edge(e, ecarry):
            av = exb[e] * R[e]
            a0 = av[0]
            a1 = av[1]
            a2 = av[2]
            a3 = av[3]
            for q in range(2):
                m = (a0 * X[e, pl.ds(q * 16, 16)]
                     + a1 * X[e, pl.ds(32 + q * 16, 16)]
                     + a2 * X[e, pl.ds(64 + q * 16, 16)]
                     + a3 * X[e, pl.ds(96 + q * 16, 16)])
                M[e, pl.ds(q * 16, 16)] = m
            return ecarry
        lax.fori_loop(0, CHUNK, _edge, 0)

        pltpu.sync_copy(M, agg_sp.at[dbuf], add=True)
        return carry
    lax.fori_loop(0, nch, _chunk, 0)

    plsc.subcore_barrier()

    def _ocp(i, carry):
        r0 = (base_z + i) * ZCH
        pltpu.sync_copy(agg_sp.at[pl.ds(r0, ZCH)],
                        agg_hbm.at[pl.ds(cN + r0, ZCH)])
        return carry
    lax.fori_loop(0, nz, _ocp, 0)


def _fin_body(agg_ref, bias_ref, gam_ref, bet_ref, y_ref):
    a = jnp.concatenate([agg_ref[0], agg_ref[1]], axis=-1) * (1.0 / H)
    a = a + bias_ref[...]
    mu = jnp.mean(a, axis=-1, keepdims=True)
    var = jnp.mean((a - mu) ** 2, axis=-1, keepdims=True)
    yn = (a - mu) / jnp.sqrt(var + 1e-5) * gam_ref[...] + bet_ref[...]
    y_ref[...] = yn * jax.nn.sigmoid(yn)


_finalize = pl.pallas_call(
    _fin_body,
    grid=(125,),
    in_specs=[
        pl.BlockSpec((NC, 400, PARTC), lambda i: (0, i, 0)),
        pl.BlockSpec((1, C), lambda i: (0, 0)),
        pl.BlockSpec((1, C), lambda i: (0, 0)),
        pl.BlockSpec((1, C), lambda i: (0, 0)),
    ],
    out_specs=pl.BlockSpec((400, C), lambda i: (i, 0)),
    out_shape=jax.ShapeDtypeStruct((N, C), jnp.float32),
)


def kernel(t, h, edge_index, edge_attr, W, We, att_src, att_dst, att_edge,
           bias, ln_gamma, ln_beta):
    del t
    f32 = jnp.float32
    Wr = W.reshape(HID, H, NC, PARTC)
    Wp = Wr.transpose(0, 2, 1, 3).reshape(HID, H * C)
    Ws = jnp.einsum("khc,hc->kh", W.reshape(HID, H, C), att_src)
    Wd = jnp.einsum("khc,hc->kh", W.reshape(HID, H, C), att_dst)
    Ae = jnp.einsum("dhc,hc->dh", We.reshape(ED, H, C), att_edge)
    Ws16 = jnp.zeros((HID, PAD), f32).at[:, :H].set(Ws)
    Wd16 = jnp.zeros((HID, PAD), f32).at[:, :H].set(Wd)
    Ae16 = jnp.zeros((ED, PAD), f32).at[:, :H].set(Ae)

    x2, asrc16, adst16 = _proj(h, Wp, Ws16, Wd16)
    ae16 = _ae_proj(edge_attr, Ae16)

    src = edge_index[0]
    dst = edge_index[1]

    ex16, dpart = _edge_ex(src, dst, asrc16, adst16, ae16)
    rden = _rden(dpart.reshape(NC, N * PAD // 128, 128)).reshape(N, PAD)
    agg = _aggregate(src, dst, x2.reshape(NC * N, 128), rden, ex16)
    y = _finalize(agg.reshape(NC, N, PARTC), bias.reshape(1, C),
                  ln_gamma.reshape(1, C), ln_beta.reshape(1, C))
    return y


# speedup vs baseline: 24.4043x; 1.1508x over previous
import functools

import jax
import jax.numpy as jnp
from jax import lax
from jax.experimental import pallas as pl
from jax.experimental.pallas import tpu as pltpu
from jax.experimental.pallas import tpu_sc as plsc

N = 50000
E = 800000
HID = 64
H = 4
C = 64
ED = 4

NC = 2
NS = 16
PARTC = C // NC
PAD = 16

CHUNK = 128
NCHUNKS = E // CHUNK
ZCH = 200
NZCH = N // ZCH

_mesh = plsc.VectorSubcoreMesh(
    core_axis_name="c", subcore_axis_name="s", num_cores=NC, num_subcores=NS)

_sc_params = pltpu.CompilerParams(use_tc_tiling_on_sc=False)


def _proj_body(h_ref, wp_ref, ws_ref, wd_ref, x2_ref, asrc_ref, adst_ref):
    hb = h_ref[...]
    xb = jnp.dot(hb, wp_ref[...], preferred_element_type=jnp.float32)
    x2_ref[0] = xb[:, :128]
    x2_ref[1] = xb[:, 128:]
    asrc_ref[...] = jnp.dot(hb, ws_ref[...], preferred_element_type=jnp.float32)
    adst_ref[...] = jnp.dot(hb, wd_ref[...], preferred_element_type=jnp.float32)


_proj = pl.pallas_call(
    _proj_body,
    grid=(125,),
    in_specs=[
        pl.BlockSpec((400, HID), lambda i: (i, 0)),
        pl.BlockSpec((HID, H * C), lambda i: (0, 0)),
        pl.BlockSpec((HID, PAD), lambda i: (0, 0)),
        pl.BlockSpec((HID, PAD), lambda i: (0, 0)),
    ],
    out_specs=[
        pl.BlockSpec((NC, 400, 128), lambda i: (0, i, 0)),
        pl.BlockSpec((400, PAD), lambda i: (i, 0)),
        pl.BlockSpec((400, PAD), lambda i: (i, 0)),
    ],
    out_shape=[
        jax.ShapeDtypeStruct((NC, N, 128), jnp.float32),
        jax.ShapeDtypeStruct((N, PAD), jnp.float32),
        jax.ShapeDtypeStruct((N, PAD), jnp.float32),
    ],
)


def _ae_body(ea_ref, w_ref, ae_ref):
    ea = ea_ref[...]
    w = w_ref[...]
    acc = ea[:, 0:1] * w[0:1, :]
    for d in range(1, ED):
        acc = acc + ea[:, d:d + 1] * w[d:d + 1, :]
    ae_ref[...] = acc


_ae_proj = pl.pallas_call(
    _ae_body,
    grid=(500,),
    in_specs=[
        pl.BlockSpec((1600, ED), lambda i: (i, 0)),
        pl.BlockSpec((ED, PAD), lambda i: (0, 0)),
    ],
    out_specs=pl.BlockSpec((1600, PAD), lambda i: (i, 0)),
    out_shape=jax.ShapeDtypeStruct((E, PAD), jnp.float32),
)


@functools.partial(
    pl.kernel,
    mesh=_mesh,
    out_type=(
        jax.ShapeDtypeStruct((E, PAD), jnp.float32),
        jax.ShapeDtypeStruct((NC * N, PAD), jnp.float32),
    ),
    scratch_types=[
        pltpu.VMEM((CHUNK, PAD), jnp.float32),
        pltpu.VMEM((CHUNK, PAD), jnp.float32),
        pltpu.VMEM((CHUNK, PAD), jnp.float32),
        pltpu.VMEM((CHUNK, PAD), jnp.float32),
        pltpu.VMEM((CHUNK,), jnp.int32),
        pltpu.VMEM((CHUNK,), jnp.int32),
        pltpu.VMEM((ZCH, PAD), jnp.float32),
        pltpu.VMEM_SHARED((N, PAD), jnp.float32),
    ],
    compiler_params=_sc_params,
)
def _edge_ex(src_hbm, dst_hbm, asrc_hbm, adst_hbm, ae_hbm,
             ex_hbm, dpart_hbm,
             As, Ad, aeb, exb, sbuf, dbuf, zb, den_sp):
    c = lax.axis_index("c")
    s = lax.axis_index("s")
    wid = s * NC + c

    def _zrow(r, carry):
        zb[r] = jnp.zeros((PAD,), jnp.float32)
        return carry
    lax.fori_loop(0, ZCH, _zrow, 0)

    nz = jnp.where(s < 10, 16, 15)
    base_z = s * 15 + jnp.minimum(s, 10)

    def _zcp(i, carry):
        pltpu.sync_copy(zb, den_sp.at[pl.ds((base_z + i) * ZCH, ZCH)])
        return carry
    lax.fori_loop(0, nz, _zcp, 0)
    plsc.subcore_barrier()

    nch = jnp.where(wid < 10, 196, 195)
    base_ch = wid * 195 + jnp.minimum(wid, 10)

    def _chunk(i, carry):
        base = (base_ch + i) * CHUNK
        pltpu.sync_copy(src_hbm.at[pl.ds(base, CHUNK)], sbuf)
        pltpu.sync_copy(dst_hbm.at[pl.ds(base, CHUNK)], dbuf)
        pltpu.sync_copy(ae_hbm.at[pl.ds(base, CHUNK)], aeb)
        pltpu.sync_copy(asrc_hbm.at[sbuf], As)
        pltpu.sync_copy(adst_hbm.at[dbuf], Ad)

        def _edge(e, ecarry):
            sv = As[e] + Ad[e] + aeb[e]
            sv = jnp.maximum(sv, 0.2 * sv)
            exb[e] = jnp.exp(sv)
            return ecarry
        lax.fori_loop(0, CHUNK, _edge, 0)

        pltpu.sync_copy(exb, ex_hbm.at[pl.ds(base, CHUNK)])
        pltpu.sync_copy(exb, den_sp.at[dbuf], add=True)
        return carry
    lax.fori_loop(0, nch, _chunk, 0)

    plsc.subcore_barrier()

    def _ocp(i, carry):
        r0 = (base_z + i) * ZCH
        pltpu.sync_copy(den_sp.at[pl.ds(r0, ZCH)],
                        dpart_hbm.at[pl.ds(c * N + r0, ZCH)])
        return carry
    lax.fori_loop(0, nz, _ocp, 0)


def _rden_body(dp_ref, rden_ref):
    rden_ref[...] = 1.0 / (dp_ref[0] + dp_ref[1] + 1e-16)


_rden = pl.pallas_call(
    _rden_body,
    grid=(1,),
    in_specs=[pl.BlockSpec((NC, N * PAD // 128, 128), lambda i: (0, 0, 0))],
    out_specs=pl.BlockSpec((N * PAD // 128, 128), lambda i: (0, 0)),
    out_shape=jax.ShapeDtypeStruct((N * PAD // 128, 128), jnp.float32),
)


DCH = 64
DPAIRS = E // (2 * DCH)


@functools.partial(
    pl.kernel,
    mesh=_mesh,
    out_type=jax.ShapeDtypeStruct((NC * N, PARTC), jnp.float32),
    scratch_types=[
        pltpu.VMEM((2, DCH, 128), jnp.float32),
        pltpu.VMEM((2, DCH, PARTC), jnp.float32),
        pltpu.VMEM((2, DCH, PAD), jnp.float32),
        pltpu.VMEM((2, DCH, PAD), jnp.float32),
        pltpu.VMEM((2, DCH), jnp.int32),
        pltpu.VMEM((2, DCH), jnp.int32),
        pltpu.VMEM((2, DCH), jnp.int32),
        pltpu.VMEM((2, DCH), jnp.int32),
        pltpu.SemaphoreType.DMA((2,)),
        pltpu.SemaphoreType.DMA((2,)),
        pltpu.SemaphoreType.DMA((2,)),
        pltpu.VMEM_SHARED((N, PARTC), jnp.float32),
    ],
    compiler_params=_sc_params,
)
def _aggregate(src_hbm, dst_hbm, x2_hbm, rden_hbm, ex_hbm, zeros_hbm,
               agg_hbm,
               X, M, R, exb, sbuf, dbuf, ibuf, dsc,
               sem_i, sem_g, sem_s, agg_sp):
    c = lax.axis_index("c")
    s = lax.axis_index("s")

    nz = jnp.where(s < 10, 16, 15)
    base_z = s * 15 + jnp.minimum(s, 10)

    def _zcp(i, carry):
        pltpu.sync_copy(zeros_hbm, agg_sp.at[pl.ds((base_z + i) * ZCH, ZCH)])
        return carry
    lax.fori_loop(0, nz, _zcp, 0)
    plsc.subcore_barrier()

    npair = jnp.where(s < 10, 391, 390)
    base_pair = s * 390 + jnp.minimum(s, 10)
    T = 2 * npair
    cN = c * N

    def _issue_idx(k, b):
        base = (2 * base_pair + k) * DCH
        pltpu.async_copy(src_hbm.at[pl.ds(base, DCH)], sbuf.at[b], sem_i.at[b])
        pltpu.async_copy(dst_hbm.at[pl.ds(base, DCH)], dbuf.at[b], sem_i.at[b])

    def _wait_idx(b):
        pltpu.make_async_copy(src_hbm.at[pl.ds(0, DCH)], sbuf.at[b],
                              sem_i.at[b]).wait()
        pltpu.make_async_copy(dst_hbm.at[pl.ds(0, DCH)], dbuf.at[b],
                              sem_i.at[b]).wait()

    def _issue_gathers(k, b):
        for g in range(DCH // 16):
            ibuf[b, pl.ds(g * 16, 16)] = sbuf[b, pl.ds(g * 16, 16)] + cN
        base = (2 * base_pair + k) * DCH
        pltpu.async_copy(x2_hbm.at[ibuf.at[b]], X.at[b], sem_g.at[b])
        pltpu.async_copy(rden_hbm.at[dbuf.at[b]], R.at[b], sem_g.at[b])
        pltpu.async_copy(ex_hbm.at[pl.ds(base, DCH)], exb.at[b], sem_g.at[b])

    def _wait_gathers(b):
        pltpu.make_async_copy(x2_hbm.at[ibuf.at[b]], X.at[b],
                              sem_g.at[b]).wait()
        pltpu.make_async_copy(rden_hbm.at[dbuf.at[b]], R.at[b],
                              sem_g.at[b]).wait()
        pltpu.make_async_copy(ex_hbm.at[pl.ds(0, DCH)], exb.at[b],
                              sem_g.at[b]).wait()

    def _issue_scatter(b):
        pltpu.async_copy(M.at[b], agg_sp.at[dsc.at[b]], sem_s.at[b], add=True)

    def _wait_scatter(b):
        pltpu.make_async_copy(M.at[b], agg_sp.at[dsc.at[b]],
                              sem_s.at[b]).wait()

    def _compute(b):
        def _edge(e, ecarry):
            av = exb[b, e] * R[b, e]
            a0 = av[0]
            a1 = av[1]
            a2 = av[2]
            a3 = av[3]
            for q in range(2):
                m = (a0 * X[b, e, pl.ds(q * 16, 16)]
                     + a1 * X[b, e, pl.ds(32 + q * 16, 16)]
                     + a2 * X[b, e, pl.ds(64 + q * 16, 16)]
                     + a3 * X[b, e, pl.ds(96 + q * 16, 16)])
                M[b, e, pl.ds(q * 16, 16)] = m
            return ecarry
        lax.fori_loop(0, DCH, _edge, 0)
        for g in range(DCH // 16):
            dsc[b, pl.ds(g * 16, 16)] = dbuf[b, pl.ds(g * 16, 16)]

    _issue_idx(0, 0)
    _issue_idx(1, 1)
    _wait_idx(0)
    _issue_gathers(0, 0)

    def _pair(j, carry):
        for b in (0, 1):
            k = 2 * j + b
            nb = 1 - b
            _wait_gathers(b)

            @pl.when(k >= 2)
            def _():
                _wait_scatter(b)

            _compute(b)
            _issue_scatter(b)

            @pl.when(k + 1 < T)
            def _():
                _wait_idx(nb)
                _issue_gathers(k + 1, nb)

            @pl.when(k + 2 < T)
            def _():
                _issue_idx(k + 2, b)
        return carry
    lax.fori_loop(0, npair, _pair, 0)
    _wait_scatter(0)
    _wait_scatter(1)

    plsc.subcore_barrier()

    def _ocp(i, carry):
        r0 = (base_z + i) * ZCH
        pltpu.sync_copy(agg_sp.at[pl.ds(r0, ZCH)],
                        agg_hbm.at[pl.ds(cN + r0, ZCH)])
        return carry
    lax.fori_loop(0, nz, _ocp, 0)


def _fin_body(agg_ref, bias_ref, gam_ref, bet_ref, y_ref):
    a = jnp.concatenate([agg_ref[0], agg_ref[1]], axis=-1) * (1.0 / H)
    a = a + bias_ref[...]
    mu = jnp.mean(a, axis=-1, keepdims=True)
    var = jnp.mean((a - mu) ** 2, axis=-1, keepdims=True)
    yn = (a - mu) / jnp.sqrt(var + 1e-5) * gam_ref[...] + bet_ref[...]
    y_ref[...] = yn * jax.nn.sigmoid(yn)


_finalize = pl.pallas_call(
    _fin_body,
    grid=(125,),
    in_specs=[
        pl.BlockSpec((NC, 400, PARTC), lambda i: (0, i, 0)),
        pl.BlockSpec((1, C), lambda i: (0, 0)),
        pl.BlockSpec((1, C), lambda i: (0, 0)),
        pl.BlockSpec((1, C), lambda i: (0, 0)),
    ],
    out_specs=pl.BlockSpec((400, C), lambda i: (i, 0)),
    out_shape=jax.ShapeDtypeStruct((N, C), jnp.float32),
)


def kernel(t, h, edge_index, edge_attr, W, We, att_src, att_dst, att_edge,
           bias, ln_gamma, ln_beta):
    del t
    f32 = jnp.float32
    Wr = W.reshape(HID, H, NC, PARTC)
    Wp = Wr.transpose(0, 2, 1, 3).reshape(HID, H * C)
    Ws = jnp.einsum("khc,hc->kh", W.reshape(HID, H, C), att_src)
    Wd = jnp.einsum("khc,hc->kh", W.reshape(HID, H, C), att_dst)
    Ae = jnp.einsum("dhc,hc->dh", We.reshape(ED, H, C), att_edge)
    Ws16 = jnp.zeros((HID, PAD), f32).at[:, :H].set(Ws)
    Wd16 = jnp.zeros((HID, PAD), f32).at[:, :H].set(Wd)
    Ae16 = jnp.zeros((ED, PAD), f32).at[:, :H].set(Ae)

    x2, asrc16, adst16 = _proj(h, Wp, Ws16, Wd16)
    ae16 = _ae_proj(edge_attr, Ae16)

    src = edge_index[0]
    dst = edge_index[1]

    ex16, dpart = _edge_ex(src, dst, asrc16, adst16, ae16)
    rden = _rden(dpart.reshape(NC, N * PAD // 128, 128)).reshape(N, PAD)
    zeros_d = jnp.zeros((ZCH, PARTC), f32)
    agg = _aggregate(src, dst, x2.reshape(NC * N, 128), rden, ex16, zeros_d)
    y = _finalize(agg.reshape(NC, N, PARTC), bias.reshape(1, C),
                  ln_gamma.reshape(1, C), ln_beta.reshape(1, C))
    return y


# speedup vs baseline: 27.7053x; 1.1353x over previous
import functools

import jax
import jax.numpy as jnp
from jax import lax
from jax.experimental import pallas as pl
from jax.experimental.pallas import tpu as pltpu
from jax.experimental.pallas import tpu_sc as plsc

N = 50000
E = 800000
HID = 64
H = 4
C = 64
ED = 4

NC = 2
NS = 16
PARTC = C // NC
PAD = 16

CHUNK = 128
NCHUNKS = E // CHUNK
ZCH = 200
NZCH = N // ZCH

_mesh = plsc.VectorSubcoreMesh(
    core_axis_name="c", subcore_axis_name="s", num_cores=NC, num_subcores=NS)

_sc_params = pltpu.CompilerParams(use_tc_tiling_on_sc=False)


def _proj_body(h_ref, wp_ref, ws_ref, wd_ref, x2_ref, asrc_ref, adst_ref):
    hb = h_ref[...]
    xb = jnp.dot(hb, wp_ref[...], preferred_element_type=jnp.float32)
    x2_ref[0] = xb[:, :128]
    x2_ref[1] = xb[:, 128:]
    asrc_ref[...] = jnp.dot(hb, ws_ref[...], preferred_element_type=jnp.float32)
    adst_ref[...] = jnp.dot(hb, wd_ref[...], preferred_element_type=jnp.float32)


_proj = pl.pallas_call(
    _proj_body,
    grid=(125,),
    in_specs=[
        pl.BlockSpec((400, HID), lambda i: (i, 0)),
        pl.BlockSpec((HID, H * C), lambda i: (0, 0)),
        pl.BlockSpec((HID, PAD), lambda i: (0, 0)),
        pl.BlockSpec((HID, PAD), lambda i: (0, 0)),
    ],
    out_specs=[
        pl.BlockSpec((NC, 400, 128), lambda i: (0, i, 0)),
        pl.BlockSpec((400, PAD), lambda i: (i, 0)),
        pl.BlockSpec((400, PAD), lambda i: (i, 0)),
    ],
    out_shape=[
        jax.ShapeDtypeStruct((NC, N, 128), jnp.float32),
        jax.ShapeDtypeStruct((N, PAD), jnp.float32),
        jax.ShapeDtypeStruct((N, PAD), jnp.float32),
    ],
)


def _ae_body(ea_ref, w_ref, ae_ref):
    ea = ea_ref[...]
    w = w_ref[...]
    acc = ea[:, 0:1] * w[0:1, :]
    for d in range(1, ED):
        acc = acc + ea[:, d:d + 1] * w[d:d + 1, :]
    ae_ref[...] = acc


_ae_proj = pl.pallas_call(
    _ae_body,
    grid=(500,),
    in_specs=[
        pl.BlockSpec((1600, ED), lambda i: (i, 0)),
        pl.BlockSpec((ED, PAD), lambda i: (0, 0)),
    ],
    out_specs=pl.BlockSpec((1600, PAD), lambda i: (i, 0)),
    out_shape=jax.ShapeDtypeStruct((E, PAD), jnp.float32),
)


@functools.partial(
    pl.kernel,
    mesh=_mesh,
    out_type=(
        jax.ShapeDtypeStruct((E, PAD), jnp.float32),
        jax.ShapeDtypeStruct((NC * N, PAD), jnp.float32),
    ),
    scratch_types=[
        pltpu.VMEM((CHUNK, PAD), jnp.float32),
        pltpu.VMEM((CHUNK, PAD), jnp.float32),
        pltpu.VMEM((CHUNK, PAD), jnp.float32),
        pltpu.VMEM((CHUNK, PAD), jnp.float32),
        pltpu.VMEM((CHUNK,), jnp.int32),
        pltpu.VMEM((CHUNK,), jnp.int32),
        pltpu.VMEM((ZCH, PAD), jnp.float32),
        pltpu.VMEM_SHARED((N, PAD), jnp.float32),
    ],
    compiler_params=_sc_params,
)
def _edge_ex(src_hbm, dst_hbm, asrc_hbm, adst_hbm, ae_hbm,
             ex_hbm, dpart_hbm,
             As, Ad, aeb, exb, sbuf, dbuf, zb, den_sp):
    c = lax.axis_index("c")
    s = lax.axis_index("s")
    wid = s * NC + c

    def _zrow(r, carry):
        zb[r] = jnp.zeros((PAD,), jnp.float32)
        return carry
    lax.fori_loop(0, ZCH, _zrow, 0)

    nz = jnp.where(s < 10, 16, 15)
    base_z = s * 15 + jnp.minimum(s, 10)

    def _zcp(i, carry):
        pltpu.sync_copy(zb, den_sp.at[pl.ds((base_z + i) * ZCH, ZCH)])
        return carry
    lax.fori_loop(0, nz, _zcp, 0)
    plsc.subcore_barrier()

    nch = jnp.where(wid < 10, 196, 195)
    base_ch = wid * 195 + jnp.minimum(wid, 10)

    def _chunk(i, carry):
        base = (base_ch + i) * CHUNK
        pltpu.sync_copy(src_hbm.at[pl.ds(base, CHUNK)], sbuf)
        pltpu.sync_copy(dst_hbm.at[pl.ds(base, CHUNK)], dbuf)
        pltpu.sync_copy(ae_hbm.at[pl.ds(base, CHUNK)], aeb)
        pltpu.sync_copy(asrc_hbm.at[sbuf], As)
        pltpu.sync_copy(adst_hbm.at[dbuf], Ad)

        def _edge(e, ecarry):
            sv = As[e] + Ad[e] + aeb[e]
            sv = jnp.maximum(sv, 0.2 * sv)
            exb[e] = jnp.exp(sv)
            return ecarry
        lax.fori_loop(0, CHUNK, _edge, 0)

        pltpu.sync_copy(exb, ex_hbm.at[pl.ds(base, CHUNK)])
        pltpu.sync_copy(exb, den_sp.at[dbuf], add=True)
        return carry
    lax.fori_loop(0, nch, _chunk, 0)

    plsc.subcore_barrier()

    def _ocp(i, carry):
        r0 = (base_z + i) * ZCH
        pltpu.sync_copy(den_sp.at[pl.ds(r0, ZCH)],
                        dpart_hbm.at[pl.ds(c * N + r0, ZCH)])
        return carry
    lax.fori_loop(0, nz, _ocp, 0)


def _rden_body(dp_ref, rden_ref):
    rden_ref[...] = 1.0 / (dp_ref[0] + dp_ref[1] + 1e-16)


_rden = pl.pallas_call(
    _rden_body,
    grid=(1,),
    in_specs=[pl.BlockSpec((NC, N * PAD // 128, 128), lambda i: (0, 0, 0))],
    out_specs=pl.BlockSpec((N * PAD // 128, 128), lambda i: (0, 0)),
    out_shape=jax.ShapeDtypeStruct((N * PAD // 128, 128), jnp.float32),
)


DCH = 64
DPAIRS = E // (2 * DCH)


@functools.partial(
    pl.kernel,
    mesh=_mesh,
    out_type=jax.ShapeDtypeStruct((NC * N, PARTC), jnp.float32),
    scratch_types=[
        pltpu.VMEM((2, DCH, 128), jnp.float32),
        pltpu.VMEM((2, DCH, PARTC), jnp.float32),
        pltpu.VMEM((2, DCH, PAD), jnp.float32),
        pltpu.VMEM((2, DCH, PAD), jnp.float32),
        pltpu.VMEM((2, DCH), jnp.int32),
        pltpu.VMEM((2, DCH), jnp.int32),
        pltpu.VMEM((2, DCH), jnp.int32),
        pltpu.VMEM((2, DCH), jnp.int32),
        pltpu.SemaphoreType.DMA((2,)),
        pltpu.SemaphoreType.DMA((2,)),
        pltpu.SemaphoreType.DMA((2,)),
        pltpu.VMEM_SHARED((N, PARTC), jnp.float32),
    ],
    compiler_params=_sc_params,
)
def _aggregate(src_hbm, dst_hbm, x2_hbm, rden_hbm, ex_hbm, zeros_hbm,
               agg_hbm,
               X, M, R, exb, sbuf, dbuf, ibuf, dsc,
               sem_i, sem_g, sem_s, agg_sp):
    c = lax.axis_index("c")
    s = lax.axis_index("s")

    nz = jnp.where(s < 10, 16, 15)
    base_z = s * 15 + jnp.minimum(s, 10)

    def _zcp(i, carry):
        pltpu.sync_copy(zeros_hbm, agg_sp.at[pl.ds((base_z + i) * ZCH, ZCH)])
        return carry
    lax.fori_loop(0, nz, _zcp, 0)
    plsc.subcore_barrier()

    npair = jnp.where(s < 10, 391, 390)
    base_pair = s * 390 + jnp.minimum(s, 10)
    T = 2 * npair
    cN = c * N

    def _issue_idx(k, b):
        base = (2 * base_pair + k) * DCH
        pltpu.async_copy(src_hbm.at[pl.ds(base, DCH)], sbuf.at[b], sem_i.at[b])
        pltpu.async_copy(dst_hbm.at[pl.ds(base, DCH)], dbuf.at[b], sem_i.at[b])

    def _wait_idx(b):
        pltpu.make_async_copy(src_hbm.at[pl.ds(0, DCH)], sbuf.at[b],
                              sem_i.at[b]).wait()
        pltpu.make_async_copy(dst_hbm.at[pl.ds(0, DCH)], dbuf.at[b],
                              sem_i.at[b]).wait()

    def _issue_gathers(k, b):
        for g in range(DCH // 16):
            ibuf[b, pl.ds(g * 16, 16)] = sbuf[b, pl.ds(g * 16, 16)] + cN
        base = (2 * base_pair + k) * DCH
        pltpu.async_copy(x2_hbm.at[ibuf.at[b]], X.at[b], sem_g.at[b])
        pltpu.async_copy(rden_hbm.at[dbuf.at[b]], R.at[b], sem_g.at[b])
        pltpu.async_copy(ex_hbm.at[pl.ds(base, DCH)], exb.at[b], sem_g.at[b])

    def _wait_gathers(b):
        pltpu.make_async_copy(x2_hbm.at[ibuf.at[b]], X.at[b],
                              sem_g.at[b]).wait()
        pltpu.make_async_copy(rden_hbm.at[dbuf.at[b]], R.at[b],
                              sem_g.at[b]).wait()
        pltpu.make_async_copy(ex_hbm.at[pl.ds(0, DCH)], exb.at[b],
                              sem_g.at[b]).wait()

    def _issue_scatter(b):
        pltpu.async_copy(M.at[b], agg_sp.at[dsc.at[b]], sem_s.at[b], add=True)

    def _wait_scatter(b):
        pltpu.make_async_copy(M.at[b], agg_sp.at[dsc.at[b]],
                              sem_s.at[b]).wait()

    def _compute(b):
        def _edge(e, ecarry):
            av = exb[b, e] * R[b, e]
            a0 = av[0]
            a1 = av[1]
            a2 = av[2]
            a3 = av[3]
            for q in range(2):
                m = (a0 * X[b, e, pl.ds(q * 16, 16)]
                     + a1 * X[b, e, pl.ds(32 + q * 16, 16)]
                     + a2 * X[b, e, pl.ds(64 + q * 16, 16)]
                     + a3 * X[b, e, pl.ds(96 + q * 16, 16)])
                M[b, e, pl.ds(q * 16, 16)] = m
            return ecarry
        lax.fori_loop(0, DCH, _edge, 0, unroll=2)
        for g in range(DCH // 16):
            dsc[b, pl.ds(g * 16, 16)] = dbuf[b, pl.ds(g * 16, 16)]

    _issue_idx(0, 0)
    _issue_idx(1, 1)
    _wait_idx(0)
    _issue_gathers(0, 0)

    def _pair(j, carry):
        for b in (0, 1):
            k = 2 * j + b
            nb = 1 - b
            _wait_gathers(b)

            @pl.when(k + 1 < T)
            def _():
                _wait_idx(nb)
                _issue_gathers(k + 1, nb)

            @pl.when(k >= 2)
            def _():
                _wait_scatter(b)

            _compute(b)
            _issue_scatter(b)

            @pl.when(k + 2 < T)
            def _():
                _issue_idx(k + 2, b)
        return carry
    lax.fori_loop(0, npair, _pair, 0)
    _wait_scatter(0)
    _wait_scatter(1)

    plsc.subcore_barrier()

    def _ocp(i, carry):
        r0 = (base_z + i) * ZCH
        pltpu.sync_copy(agg_sp.at[pl.ds(r0, ZCH)],
                        agg_hbm.at[pl.ds(cN + r0, ZCH)])
        return carry
    lax.fori_loop(0, nz, _ocp, 0)


def _fin_body(agg_ref, bias_ref, gam_ref, bet_ref, y_ref):
    a = jnp.concatenate([agg_ref[0], agg_ref[1]], axis=-1) * (1.0 / H)
    a = a + bias_ref[...]
    mu = jnp.mean(a, axis=-1, keepdims=True)
    var = jnp.mean((a - mu) ** 2, axis=-1, keepdims=True)
    yn = (a - mu) / jnp.sqrt(var + 1e-5) * gam_ref[...] + bet_ref[...]
    y_ref[...] = yn * jax.nn.sigmoid(yn)


_finalize = pl.pallas_call(
    _fin_body,
    grid=(125,),
    in_specs=[
        pl.BlockSpec((NC, 400, PARTC), lambda i: (0, i, 0)),
        pl.BlockSpec((1, C), lambda i: (0, 0)),
        pl.BlockSpec((1, C), lambda i: (0, 0)),
        pl.BlockSpec((1, C), lambda i: (0, 0)),
    ],
    out_specs=pl.BlockSpec((400, C), lambda i: (i, 0)),
    out_shape=jax.ShapeDtypeStruct((N, C), jnp.float32),
)


def kernel(t, h, edge_index, edge_attr, W, We, att_src, att_dst, att_edge,
           bias, ln_gamma, ln_beta):
    del t
    f32 = jnp.float32
    Wr = W.reshape(HID, H, NC, PARTC)
    Wp = Wr.transpose(0, 2, 1, 3).reshape(HID, H * C)
    Ws = jnp.einsum("khc,hc->kh", W.reshape(HID, H, C), att_src)
    Wd = jnp.einsum("khc,hc->kh", W.reshape(HID, H, C), att_dst)
    Ae = jnp.einsum("dhc,hc->dh", We.reshape(ED, H, C), att_edge)
    Ws16 = jnp.zeros((HID, PAD), f32).at[:, :H].set(Ws)
    Wd16 = jnp.zeros((HID, PAD), f32).at[:, :H].set(Wd)
    Ae16 = jnp.zeros((ED, PAD), f32).at[:, :H].set(Ae)

    x2, asrc16, adst16 = _proj(h, Wp, Ws16, Wd16)
    ae16 = _ae_proj(edge_attr, Ae16)

    src = edge_index[0]
    dst = edge_index[1]

    ex16, dpart = _edge_ex(src, dst, asrc16, adst16, ae16)
    rden = _rden(dpart.reshape(NC, N * PAD // 128, 128)).reshape(N, PAD)
    zeros_d = jnp.zeros((ZCH, PARTC), f32)
    agg = _aggregate(src, dst, x2.reshape(NC * N, 128), rden, ex16, zeros_d)
    y = _finalize(agg.reshape(NC, N, PARTC), bias.reshape(1, C),
                  ln_gamma.reshape(1, C), ln_beta.reshape(1, C))
    return y
